# Initial kernel scaffold; baseline (speedup 1.0000x reference)
import jax
import jax.numpy as jnp
from jax.experimental import pallas as pl


def kernel(x, edge_index, edge_attr, target_indices, W0, b0, Wn1, bn1, Wn2, bn2, conv_b, W_ih, W_hh, b_ih, b_hh, W1, b1, W2, b2):
    raise NotImplementedError("write your pallas kernel here")


# speedup vs baseline: 1.1737x; 1.1737x over previous
import functools

import jax
import jax.numpy as jnp
from jax import lax
from jax.experimental import pallas as pl
from jax.experimental.pallas import tpu as pltpu
from jax.experimental.pallas import tpu_sc as plsc

N = 10000
E = 160000
F_IN = 128
DIM = 16

NC = 2
NS = 16
NW = NC * NS

CH = 128
FIRE = 8
SUP = CH * FIRE
CPW = 40
NSUP = CPW // FIRE
EP = NW * CPW * CH
NP = N + 16
RPS = NP // NS

_mesh = functools.partial(
    plsc.VectorSubcoreMesh,
    core_axis_name="c", subcore_axis_name="s", num_cores=NC, num_subcores=NS,
)


def _wid():
  return lax.axis_index("s") * NC + lax.axis_index("c")


@functools.partial(
    pl.kernel,
    out_type=jax.ShapeDtypeStruct((EP, DIM), jnp.float32),
    mesh=_mesh(),
    compiler_params=pltpu.CompilerParams(use_tc_tiling_on_sc=False),
    scratch_types=[
        pltpu.VMEM((FIRE, CH), jnp.int32),
        pltpu.VMEM((SUP, DIM), jnp.float32),
        pltpu.SemaphoreType.DMA,
    ],
)
def _sc_gather_edges(table_hbm, idx_hbm, out_hbm, idx_v, rows_v, sem):
  w = _wid()

  def body(s, carry):
    chunk0 = w * CPW + s * FIRE
    base = chunk0 * CH
    pltpu.sync_copy(idx_hbm.at[pl.ds(chunk0, FIRE)], idx_v)
    copies = [
        pltpu.async_copy(table_hbm.at[idx_v.at[b]],
                         rows_v.at[pl.ds(b * CH, CH)], sem)
        for b in range(FIRE)
    ]
    for c in copies:
      c.wait()
    pltpu.sync_copy(rows_v, out_hbm.at[pl.ds(base, SUP)])
    return carry

  lax.fori_loop(0, NSUP, body, 0)


@functools.partial(
    pl.kernel,
    out_type=jax.ShapeDtypeStruct((NC, NP, DIM), jnp.float32),
    mesh=_mesh(),
    compiler_params=pltpu.CompilerParams(use_tc_tiling_on_sc=False),
    scratch_types=[
        pltpu.VMEM((FIRE, CH), jnp.int32),
        pltpu.VMEM((SUP, DIM), jnp.float32),
        pltpu.VMEM_SHARED((NP, DIM), jnp.float32),
        pltpu.SemaphoreType.DMA,
    ],
)
def _sc_scatter_add(msg_hbm, dst_hbm, zeros_hbm, out_hbm,
                    dst_v, msg_v, agg_sh, sem):
  cid = lax.axis_index("c")
  sid = lax.axis_index("s")
  w = _wid()
  rows = pl.ds(sid * RPS, RPS)
  pltpu.sync_copy(zeros_hbm, agg_sh.at[rows])
  plsc.subcore_barrier()

  def body(s, carry):
    chunk0 = w * CPW + s * FIRE
    pltpu.sync_copy(dst_hbm.at[pl.ds(chunk0, FIRE)], dst_v)
    pltpu.sync_copy(msg_hbm.at[pl.ds(chunk0 * CH, SUP)], msg_v)
    copies = [
        pltpu.async_copy(msg_v.at[pl.ds(b * CH, CH)],
                         agg_sh.at[dst_v.at[b]], sem, add=True)
        for b in range(FIRE)
    ]
    for c in copies:
      c.wait()
    return carry

  lax.fori_loop(0, NSUP, body, 0)
  plsc.subcore_barrier()
  pltpu.sync_copy(agg_sh.at[rows], out_hbm.at[cid].at[rows])


@functools.partial(
    pl.kernel,
    out_type=jax.ShapeDtypeStruct((NC, NP, DIM), jnp.float32),
    mesh=_mesh(),
    compiler_params=pltpu.CompilerParams(use_tc_tiling_on_sc=False),
    scratch_types=[
        pltpu.VMEM((FIRE, CH), jnp.int32),
        pltpu.VMEM((CH, DIM), jnp.float32),
        pltpu.VMEM_SHARED((NP, DIM), jnp.float32),
        pltpu.SemaphoreType.DMA,
    ],
)
def _sc_degree(dst_hbm, ones_hbm, zeros_hbm, out_hbm,
               dst_v, ones_v, agg_sh, sem):
  cid = lax.axis_index("c")
  sid = lax.axis_index("s")
  w = _wid()
  rows = pl.ds(sid * RPS, RPS)
  pltpu.sync_copy(zeros_hbm, agg_sh.at[rows])
  pltpu.sync_copy(ones_hbm, ones_v)
  plsc.subcore_barrier()

  def body(s, carry):
    chunk0 = w * CPW + s * FIRE
    pltpu.sync_copy(dst_hbm.at[pl.ds(chunk0, FIRE)], dst_v)
    copies = [
        pltpu.async_copy(ones_v, agg_sh.at[dst_v.at[b]], sem, add=True)
        for b in range(FIRE)
    ]
    for c in copies:
      c.wait()
    return carry

  lax.fori_loop(0, NSUP, body, 0)
  plsc.subcore_barrier()
  pltpu.sync_copy(agg_sh.at[rows], out_hbm.at[cid].at[rows])


@functools.partial(
    pl.kernel,
    out_type=jax.ShapeDtypeStruct((1024, DIM), jnp.float32),
    mesh=_mesh(),
    compiler_params=pltpu.CompilerParams(use_tc_tiling_on_sc=False),
    scratch_types=[
        pltpu.VMEM((32,), jnp.int32),
        pltpu.VMEM((32, DIM), jnp.float32),
        pltpu.SemaphoreType.DMA,
    ],
)
def _sc_gather_targets(table_hbm, idx_hbm, out_hbm, idx_v, rows_v, sem):
  w = _wid()
  base = w * 32
  pltpu.sync_copy(idx_hbm.at[pl.ds(base, 32)], idx_v)
  pltpu.async_copy(table_hbm.at[idx_v], rows_v, sem).wait()
  pltpu.sync_copy(rows_v, out_hbm.at[pl.ds(base, 32)])


def _proj_body(x_ref, w_ref, b_ref, o_ref):
  o_ref[...] = jax.nn.relu(
      jnp.dot(x_ref[...], w_ref[...], preferred_element_type=jnp.float32,
              precision=lax.Precision.HIGHEST)
      + b_ref[...])


NB = 2000


def _tc_proj(x, w0, b0):
  full = lambda shape: pl.BlockSpec(shape, lambda i: (0,) * len(shape))
  return pl.pallas_call(
      _proj_body,
      grid=(N // NB,),
      in_specs=[pl.BlockSpec((NB, F_IN), lambda i: (i, 0)),
                full((F_IN, DIM)), full((1, DIM))],
      out_specs=pl.BlockSpec((NB, DIM), lambda i: (i, 0)),
      out_shape=jax.ShapeDtypeStruct((N, DIM), jnp.float32),
  )(x, w0, b0)


EB = 2048


def _msg_body(ea_ref, xj_ref, wn1_ref, bn1_ref, wn2_ref, bn2_ref,
              r_ref, s_ref, o_ref):
  h1 = jax.nn.relu(
      jnp.dot(ea_ref[...], wn1_ref[...], preferred_element_type=jnp.float32,
              precision=lax.Precision.HIGHEST)
      + bn1_ref[...])
  ew = jnp.dot(h1, wn2_ref[...], preferred_element_type=jnp.float32,
              precision=lax.Precision.HIGHEST)
  ew = ew + bn2_ref[...]
  xr = jnp.dot(xj_ref[...], r_ref[...], preferred_element_type=jnp.float32,
              precision=lax.Precision.HIGHEST)
  o_ref[...] = jnp.dot(xr * ew, s_ref[...],
                       preferred_element_type=jnp.float32,
              precision=lax.Precision.HIGHEST)


def _tc_msg(ea8, xj, wn1p, bn1, wn2, bn2, rmat, smat):
  grid = EP // EB
  full = lambda shape: pl.BlockSpec(shape, lambda i: (0,) * len(shape))
  return pl.pallas_call(
      _msg_body,
      grid=(grid,),
      in_specs=[
          pl.BlockSpec((EB, 8), lambda i: (i, 0)),
          pl.BlockSpec((EB, DIM), lambda i: (i, 0)),
          full((8, F_IN)),
          full((1, F_IN)),
          full((F_IN, DIM * DIM)),
          full((1, DIM * DIM)),
          full((DIM, DIM * DIM)),
          full((DIM * DIM, DIM)),
      ],
      out_specs=pl.BlockSpec((EB, DIM), lambda i: (i, 0)),
      out_shape=jax.ShapeDtypeStruct((EP, DIM), jnp.float32),
  )(ea8, xj, wn1p, bn1, wn2, bn2, rmat, smat)


def _update_body(agg_ref, deg_ref, h_ref, cb_ref,
                 wr_i, wz_i, wn_i, wr_h, wz_h, wn_h,
                 br_i, bz_i, bn_i, br_h, bz_h, bn_h, o_ref):
  agg = agg_ref[0] + agg_ref[1]
  deg = jnp.maximum(deg_ref[0] + deg_ref[1], 1.0)
  m = jax.nn.relu(agg / deg + cb_ref[...])
  h = h_ref[...]
  dot = lambda a, w: jnp.dot(a, w[...], preferred_element_type=jnp.float32,
              precision=lax.Precision.HIGHEST)
  r = jax.nn.sigmoid(dot(m, wr_i) + br_i[...] + dot(h, wr_h) + br_h[...])
  z = jax.nn.sigmoid(dot(m, wz_i) + bz_i[...] + dot(h, wz_h) + bz_h[...])
  n = jnp.tanh(dot(m, wn_i) + bn_i[...] + r * (dot(h, wn_h) + bn_h[...]))
  o_ref[...] = (1.0 - z) * n + z * h


def _tc_update(agg2, deg2, h, cb, gates):
  full = lambda shape: pl.BlockSpec(shape, lambda i: (0,) * len(shape))
  part = pl.BlockSpec((NC, NB, DIM), lambda i: (0, i, 0))
  w16 = full((DIM, DIM))
  b16 = full((1, DIM))
  return pl.pallas_call(
      _update_body,
      grid=(N // NB,),
      in_specs=[part, part,
                pl.BlockSpec((NB, DIM), lambda i: (i, 0)),
                b16, w16, w16, w16, w16, w16, w16,
                b16, b16, b16, b16, b16, b16],
      out_specs=pl.BlockSpec((NB, DIM), lambda i: (i, 0)),
      out_shape=jax.ShapeDtypeStruct((N, DIM), jnp.float32),
  )(agg2, deg2, h, cb, *gates)


def _head_body(n0_ref, w1_ref, b1_ref, w2_ref, b2_ref, o_ref):
  w1c = w1_ref[:DIM, :] + w1_ref[DIM:, :]
  p = jnp.dot(n0_ref[...], w1c, preferred_element_type=jnp.float32,
              precision=lax.Precision.HIGHEST)
  p = p + b1_ref[...]
  o_ref[...] = jnp.dot(p, w2_ref[...],
                       preferred_element_type=jnp.float32,
              precision=lax.Precision.HIGHEST) + b2_ref[...]


def _tc_head(n0, w1, b1, w2, b2):
  return pl.pallas_call(
      _head_body,
      out_shape=jax.ShapeDtypeStruct((1024, 1), jnp.float32),
  )(n0, w1, b1, w2, b2)


def kernel(x, edge_index, edge_attr, target_indices, W0, b0, Wn1, bn1, Wn2,
           bn2, conv_b, W_ih, W_hh, b_ih, b_hh, W1, b1, W2, b2):
  src = edge_index[0].astype(jnp.int32)
  dst = edge_index[1].astype(jnp.int32)
  atom0 = target_indices[0].astype(jnp.int32)

  pad = EP - E
  src2d = jnp.pad(src, (0, pad)).reshape(EP // CH, CH)
  dst2d = jnp.pad(dst, (0, pad), constant_values=N).reshape(EP // CH, CH)
  ea8 = jnp.pad(edge_attr, ((0, pad), (0, 4)))
  wn1p = jnp.pad(Wn1, ((0, 4), (0, 0)))

  ci = lax.broadcasted_iota(jnp.int32, (DIM, DIM * DIM), 1)
  ri = lax.broadcasted_iota(jnp.int32, (DIM, DIM * DIM), 0)
  rmat = (ci // DIM == ri).astype(jnp.float32)
  smat = (ci % DIM == ri).astype(jnp.float32).T

  zeros_sub = jnp.zeros((RPS, DIM), jnp.float32)
  ones_ch = jnp.ones((CH, DIM), jnp.float32)

  row = lambda v: v.reshape(1, -1)
  gates = (W_ih[0 * DIM:1 * DIM].T, W_ih[1 * DIM:2 * DIM].T,
           W_ih[2 * DIM:3 * DIM].T, W_hh[0 * DIM:1 * DIM].T,
           W_hh[1 * DIM:2 * DIM].T, W_hh[2 * DIM:3 * DIM].T,
           row(b_ih[0 * DIM:1 * DIM]), row(b_ih[1 * DIM:2 * DIM]),
           row(b_ih[2 * DIM:3 * DIM]), row(b_hh[0 * DIM:1 * DIM]),
           row(b_hh[1 * DIM:2 * DIM]), row(b_hh[2 * DIM:3 * DIM]))

  h = _tc_proj(x, W0, row(b0))
  deg2 = _sc_degree(dst2d, ones_ch, zeros_sub)
  for _ in range(3):
    xj = _sc_gather_edges(h, src2d)
    msg = _tc_msg(ea8, xj, wn1p, row(bn1), Wn2, row(bn2), rmat, smat)
    agg2 = _sc_scatter_add(msg, dst2d, zeros_sub)
    h = _tc_update(agg2, deg2, h, row(conv_b), gates)

  n0 = _sc_gather_targets(h, atom0)
  return _tc_head(n0, W1, row(b1), W2, row(b2))


# speedup vs baseline: 2.4845x; 2.1168x over previous
import functools

import jax
import jax.numpy as jnp
from jax import lax
from jax.experimental import pallas as pl
from jax.experimental.pallas import tpu as pltpu
from jax.experimental.pallas import tpu_sc as plsc

N = 10000
E = 160000
F_IN = 128
DIM = 16

NC = 2
NS = 16
NW = NC * NS

CH = 128
FIRE = 8
SUP = CH * FIRE
CPW = 40
NSUP = CPW // FIRE
EP = NW * CPW * CH
NP = N + 16
RPS = NP // NS

_mesh = functools.partial(
    plsc.VectorSubcoreMesh,
    core_axis_name="c", subcore_axis_name="s", num_cores=NC, num_subcores=NS,
)


def _wid():
  return lax.axis_index("s") * NC + lax.axis_index("c")


@functools.partial(
    pl.kernel,
    out_type=jax.ShapeDtypeStruct((EP, DIM), jnp.float32),
    mesh=_mesh(),
    compiler_params=pltpu.CompilerParams(use_tc_tiling_on_sc=False),
    scratch_types=[
        pltpu.VMEM((FIRE, CH), jnp.int32),
        pltpu.VMEM((SUP, DIM), jnp.float32),
        pltpu.SemaphoreType.DMA,
    ],
)
def _sc_gather_edges(table_hbm, idx_hbm, out_hbm, idx_v, rows_v, sem):
  w = _wid()

  def body(s, carry):
    chunk0 = w * CPW + s * FIRE
    base = chunk0 * CH
    pltpu.sync_copy(idx_hbm.at[pl.ds(chunk0, FIRE)], idx_v)
    copies = [
        pltpu.async_copy(table_hbm.at[idx_v.at[b]],
                         rows_v.at[pl.ds(b * CH, CH)], sem)
        for b in range(FIRE)
    ]
    for c in copies:
      c.wait()
    pltpu.sync_copy(rows_v, out_hbm.at[pl.ds(base, SUP)])
    return carry

  lax.fori_loop(0, NSUP, body, 0)


@functools.partial(
    pl.kernel,
    out_type=jax.ShapeDtypeStruct((NC, NP, DIM), jnp.float32),
    mesh=_mesh(),
    compiler_params=pltpu.CompilerParams(use_tc_tiling_on_sc=False),
    scratch_types=[
        pltpu.VMEM((FIRE, CH), jnp.int32),
        pltpu.VMEM((SUP, DIM), jnp.float32),
        pltpu.VMEM_SHARED((NP, DIM), jnp.float32),
        pltpu.SemaphoreType.DMA,
    ],
)
def _sc_scatter_add(msg_hbm, dst_hbm, zeros_hbm, out_hbm,
                    dst_v, msg_v, agg_sh, sem):
  cid = lax.axis_index("c")
  sid = lax.axis_index("s")
  w = _wid()
  rows = pl.ds(sid * RPS, RPS)
  pltpu.sync_copy(zeros_hbm, agg_sh.at[rows])
  plsc.subcore_barrier()

  def body(s, carry):
    chunk0 = w * CPW + s * FIRE
    pltpu.sync_copy(dst_hbm.at[pl.ds(chunk0, FIRE)], dst_v)
    pltpu.sync_copy(msg_hbm.at[pl.ds(chunk0 * CH, SUP)], msg_v)
    copies = [
        pltpu.async_copy(msg_v.at[pl.ds(b * CH, CH)],
                         agg_sh.at[dst_v.at[b]], sem, add=True)
        for b in range(FIRE)
    ]
    for c in copies:
      c.wait()
    return carry

  lax.fori_loop(0, NSUP, body, 0)
  plsc.subcore_barrier()
  pltpu.sync_copy(agg_sh.at[rows], out_hbm.at[cid].at[rows])


@functools.partial(
    pl.kernel,
    out_type=jax.ShapeDtypeStruct((NC, NP, DIM), jnp.float32),
    mesh=_mesh(),
    compiler_params=pltpu.CompilerParams(use_tc_tiling_on_sc=False),
    scratch_types=[
        pltpu.VMEM((FIRE, CH), jnp.int32),
        pltpu.VMEM((CH, DIM), jnp.float32),
        pltpu.VMEM_SHARED((NP, DIM), jnp.float32),
        pltpu.SemaphoreType.DMA,
    ],
)
def _sc_degree(dst_hbm, ones_hbm, zeros_hbm, out_hbm,
               dst_v, ones_v, agg_sh, sem):
  cid = lax.axis_index("c")
  sid = lax.axis_index("s")
  w = _wid()
  rows = pl.ds(sid * RPS, RPS)
  pltpu.sync_copy(zeros_hbm, agg_sh.at[rows])
  pltpu.sync_copy(ones_hbm, ones_v)
  plsc.subcore_barrier()

  def body(s, carry):
    chunk0 = w * CPW + s * FIRE
    pltpu.sync_copy(dst_hbm.at[pl.ds(chunk0, FIRE)], dst_v)
    copies = [
        pltpu.async_copy(ones_v, agg_sh.at[dst_v.at[b]], sem, add=True)
        for b in range(FIRE)
    ]
    for c in copies:
      c.wait()
    return carry

  lax.fori_loop(0, NSUP, body, 0)
  plsc.subcore_barrier()
  pltpu.sync_copy(agg_sh.at[rows], out_hbm.at[cid].at[rows])


@functools.partial(
    pl.kernel,
    out_type=jax.ShapeDtypeStruct((1024, DIM), jnp.float32),
    mesh=_mesh(),
    compiler_params=pltpu.CompilerParams(use_tc_tiling_on_sc=False),
    scratch_types=[
        pltpu.VMEM((32,), jnp.int32),
        pltpu.VMEM((32, DIM), jnp.float32),
        pltpu.SemaphoreType.DMA,
    ],
)
def _sc_gather_targets(table_hbm, idx_hbm, out_hbm, idx_v, rows_v, sem):
  w = _wid()
  base = w * 32
  pltpu.sync_copy(idx_hbm.at[pl.ds(base, 32)], idx_v)
  pltpu.async_copy(table_hbm.at[idx_v], rows_v, sem).wait()
  pltpu.sync_copy(rows_v, out_hbm.at[pl.ds(base, 32)])


def _proj_body(x_ref, w_ref, b_ref, o_ref):
  o_ref[...] = jax.nn.relu(
      jnp.dot(x_ref[...], w_ref[...], preferred_element_type=jnp.float32,
              precision=lax.Precision.HIGHEST)
      + b_ref[...])


NB = 2000


def _tc_proj(x, w0, b0):
  full = lambda shape: pl.BlockSpec(shape, lambda i: (0,) * len(shape))
  return pl.pallas_call(
      _proj_body,
      grid=(N // NB,),
      in_specs=[pl.BlockSpec((NB, F_IN), lambda i: (i, 0)),
                full((F_IN, DIM)), full((1, DIM))],
      out_specs=pl.BlockSpec((NB, DIM), lambda i: (i, 0)),
      out_shape=jax.ShapeDtypeStruct((N, DIM), jnp.float32),
  )(x, w0, b0)


EB = 2048


def _msg_body(ea_ref, xj_ref, wn1t_ref, bn1_ref, wn2t_ref, bn2_ref, o_ref):
  h1t = jax.nn.relu(
      jnp.dot(wn1t_ref[...], ea_ref[...], preferred_element_type=jnp.float32,
              precision=lax.Precision.HIGHEST)
      + bn1_ref[...])
  ewt = jnp.dot(wn2t_ref[...], h1t, preferred_element_type=jnp.float32,
                precision=lax.Precision.HIGHEST)
  ewt = ewt + bn2_ref[...]
  xjt = xj_ref[...].T
  msgt = xjt[0:1, :] * ewt[0:DIM, :]
  for i in range(1, DIM):
    msgt = msgt + xjt[i:i + 1, :] * ewt[DIM * i:DIM * (i + 1), :]
  o_ref[...] = msgt.T


def _tc_msg(ea8t, xj, wn1t, bn1, wn2t, bn2):
  grid = EP // EB
  full = lambda shape: pl.BlockSpec(shape, lambda i: (0,) * len(shape))
  return pl.pallas_call(
      _msg_body,
      grid=(grid,),
      in_specs=[
          pl.BlockSpec((8, EB), lambda i: (0, i)),
          pl.BlockSpec((EB, DIM), lambda i: (i, 0)),
          full((F_IN, 8)),
          full((F_IN, 1)),
          full((DIM * DIM, F_IN)),
          full((DIM * DIM, 1)),
      ],
      out_specs=pl.BlockSpec((EB, DIM), lambda i: (i, 0)),
      out_shape=jax.ShapeDtypeStruct((EP, DIM), jnp.float32),
  )(ea8t, xj, wn1t, bn1, wn2t, bn2)


def _update_body(agg_ref, deg_ref, h_ref, cb_ref,
                 wr_i, wz_i, wn_i, wr_h, wz_h, wn_h,
                 br_i, bz_i, bn_i, br_h, bz_h, bn_h, o_ref):
  agg = agg_ref[0] + agg_ref[1]
  deg = jnp.maximum(deg_ref[0] + deg_ref[1], 1.0)
  m = jax.nn.relu(agg / deg + cb_ref[...])
  h = h_ref[...]
  dot = lambda a, w: jnp.dot(a, w[...], preferred_element_type=jnp.float32,
              precision=lax.Precision.HIGHEST)
  r = jax.nn.sigmoid(dot(m, wr_i) + br_i[...] + dot(h, wr_h) + br_h[...])
  z = jax.nn.sigmoid(dot(m, wz_i) + bz_i[...] + dot(h, wz_h) + bz_h[...])
  n = jnp.tanh(dot(m, wn_i) + bn_i[...] + r * (dot(h, wn_h) + bn_h[...]))
  o_ref[...] = (1.0 - z) * n + z * h


def _tc_update(agg2, deg2, h, cb, gates):
  full = lambda shape: pl.BlockSpec(shape, lambda i: (0,) * len(shape))
  part = pl.BlockSpec((NC, NB, DIM), lambda i: (0, i, 0))
  w16 = full((DIM, DIM))
  b16 = full((1, DIM))
  return pl.pallas_call(
      _update_body,
      grid=(N // NB,),
      in_specs=[part, part,
                pl.BlockSpec((NB, DIM), lambda i: (i, 0)),
                b16, w16, w16, w16, w16, w16, w16,
                b16, b16, b16, b16, b16, b16],
      out_specs=pl.BlockSpec((NB, DIM), lambda i: (i, 0)),
      out_shape=jax.ShapeDtypeStruct((N, DIM), jnp.float32),
  )(agg2, deg2, h, cb, *gates)


def _head_body(n0_ref, w1_ref, b1_ref, w2_ref, b2_ref, o_ref):
  w1c = w1_ref[:DIM, :] + w1_ref[DIM:, :]
  p = jnp.dot(n0_ref[...], w1c, preferred_element_type=jnp.float32,
              precision=lax.Precision.HIGHEST)
  p = p + b1_ref[...]
  o_ref[...] = jnp.dot(p, w2_ref[...],
                       preferred_element_type=jnp.float32,
              precision=lax.Precision.HIGHEST) + b2_ref[...]


def _tc_head(n0, w1, b1, w2, b2):
  return pl.pallas_call(
      _head_body,
      out_shape=jax.ShapeDtypeStruct((1024, 1), jnp.float32),
  )(n0, w1, b1, w2, b2)


def kernel(x, edge_index, edge_attr, target_indices, W0, b0, Wn1, bn1, Wn2,
           bn2, conv_b, W_ih, W_hh, b_ih, b_hh, W1, b1, W2, b2):
  src = edge_index[0].astype(jnp.int32)
  dst = edge_index[1].astype(jnp.int32)
  atom0 = target_indices[0].astype(jnp.int32)

  pad = EP - E
  src2d = jnp.pad(src, (0, pad)).reshape(EP // CH, CH)
  dst2d = jnp.pad(dst, (0, pad), constant_values=N).reshape(EP // CH, CH)
  ea8t = jnp.pad(edge_attr, ((0, pad), (0, 4))).T
  wn1t = jnp.pad(Wn1, ((0, 4), (0, 0))).T
  wn2t = Wn2.T

  zeros_sub = jnp.zeros((RPS, DIM), jnp.float32)
  ones_ch = jnp.ones((CH, DIM), jnp.float32)

  row = lambda v: v.reshape(1, -1)
  gates = (W_ih[0 * DIM:1 * DIM].T, W_ih[1 * DIM:2 * DIM].T,
           W_ih[2 * DIM:3 * DIM].T, W_hh[0 * DIM:1 * DIM].T,
           W_hh[1 * DIM:2 * DIM].T, W_hh[2 * DIM:3 * DIM].T,
           row(b_ih[0 * DIM:1 * DIM]), row(b_ih[1 * DIM:2 * DIM]),
           row(b_ih[2 * DIM:3 * DIM]), row(b_hh[0 * DIM:1 * DIM]),
           row(b_hh[1 * DIM:2 * DIM]), row(b_hh[2 * DIM:3 * DIM]))

  h = _tc_proj(x, W0, row(b0))
  deg2 = _sc_degree(dst2d, ones_ch, zeros_sub)
  for _ in range(3):
    xj = _sc_gather_edges(h, src2d)
    msg = _tc_msg(ea8t, xj, wn1t, bn1.reshape(-1, 1), wn2t,
                  bn2.reshape(-1, 1))
    agg2 = _sc_scatter_add(msg, dst2d, zeros_sub)
    h = _tc_update(agg2, deg2, h, row(conv_b), gates)

  n0 = _sc_gather_targets(h, atom0)
  return _tc_head(n0, W1, row(b1), W2, row(b2))


# speedup vs baseline: 3.7087x; 1.4927x over previous
import functools

import jax
import jax.numpy as jnp
from jax import lax
from jax.experimental import pallas as pl
from jax.experimental.pallas import tpu as pltpu
from jax.experimental.pallas import tpu_sc as plsc

N = 10000
E = 160000
F_IN = 128
DIM = 16

NC = 2
NS = 16
NW = NC * NS

CH = 128
FIRE = 8
SUP = CH * FIRE
CPW = 40
NSUP = CPW // FIRE
EP = NW * CPW * CH
NP = N + 16
RPS = NP // NS

_mesh = functools.partial(
    plsc.VectorSubcoreMesh,
    core_axis_name="c", subcore_axis_name="s", num_cores=NC, num_subcores=NS,
)


def _wid():
  return lax.axis_index("s") * NC + lax.axis_index("c")


@functools.partial(
    pl.kernel,
    out_type=jax.ShapeDtypeStruct((EP, DIM), jnp.float32),
    mesh=_mesh(),
    compiler_params=pltpu.CompilerParams(use_tc_tiling_on_sc=False),
    scratch_types=[
        pltpu.VMEM((FIRE, CH), jnp.int32),
        pltpu.VMEM((SUP, DIM), jnp.float32),
        pltpu.SemaphoreType.DMA,
    ],
)
def _sc_gather_edges(table_hbm, idx_hbm, out_hbm, idx_v, rows_v, sem):
  w = _wid()

  def body(s, carry):
    chunk0 = w * CPW + s * FIRE
    base = chunk0 * CH
    pltpu.sync_copy(idx_hbm.at[pl.ds(chunk0, FIRE)], idx_v)
    copies = [
        pltpu.async_copy(table_hbm.at[idx_v.at[b]],
                         rows_v.at[pl.ds(b * CH, CH)], sem)
        for b in range(FIRE)
    ]
    for c in copies:
      c.wait()
    pltpu.sync_copy(rows_v, out_hbm.at[pl.ds(base, SUP)])
    return carry

  lax.fori_loop(0, NSUP, body, 0)


@functools.partial(
    pl.kernel,
    out_type=jax.ShapeDtypeStruct((NC, NP, DIM), jnp.float32),
    mesh=_mesh(),
    compiler_params=pltpu.CompilerParams(use_tc_tiling_on_sc=False),
    scratch_types=[
        pltpu.VMEM((FIRE, CH), jnp.int32),
        pltpu.VMEM((SUP, DIM), jnp.float32),
        pltpu.VMEM_SHARED((NP, DIM), jnp.float32),
        pltpu.SemaphoreType.DMA,
    ],
)
def _sc_scatter_add(msg_hbm, dst_hbm, zeros_hbm, out_hbm,
                    dst_v, msg_v, agg_sh, sem):
  cid = lax.axis_index("c")
  sid = lax.axis_index("s")
  w = _wid()
  rows = pl.ds(sid * RPS, RPS)
  pltpu.sync_copy(zeros_hbm, agg_sh.at[rows])
  plsc.subcore_barrier()

  def body(s, carry):
    chunk0 = w * CPW + s * FIRE
    pltpu.sync_copy(dst_hbm.at[pl.ds(chunk0, FIRE)], dst_v)
    pltpu.sync_copy(msg_hbm.at[pl.ds(chunk0 * CH, SUP)], msg_v)
    copies = [
        pltpu.async_copy(msg_v.at[pl.ds(b * CH, CH)],
                         agg_sh.at[dst_v.at[b]], sem, add=True)
        for b in range(FIRE)
    ]
    for c in copies:
      c.wait()
    return carry

  lax.fori_loop(0, NSUP, body, 0)
  plsc.subcore_barrier()
  pltpu.sync_copy(agg_sh.at[rows], out_hbm.at[cid].at[rows])


@functools.partial(
    pl.kernel,
    out_type=jax.ShapeDtypeStruct((NC, NP, DIM), jnp.float32),
    mesh=_mesh(),
    compiler_params=pltpu.CompilerParams(use_tc_tiling_on_sc=False),
    scratch_types=[
        pltpu.VMEM((FIRE, CH), jnp.int32),
        pltpu.VMEM((CH, DIM), jnp.float32),
        pltpu.VMEM_SHARED((NP, DIM), jnp.float32),
        pltpu.SemaphoreType.DMA,
    ],
)
def _sc_degree(dst_hbm, ones_hbm, zeros_hbm, out_hbm,
               dst_v, ones_v, agg_sh, sem):
  cid = lax.axis_index("c")
  sid = lax.axis_index("s")
  w = _wid()
  rows = pl.ds(sid * RPS, RPS)
  pltpu.sync_copy(zeros_hbm, agg_sh.at[rows])
  pltpu.sync_copy(ones_hbm, ones_v)
  plsc.subcore_barrier()

  def body(s, carry):
    chunk0 = w * CPW + s * FIRE
    pltpu.sync_copy(dst_hbm.at[pl.ds(chunk0, FIRE)], dst_v)
    copies = [
        pltpu.async_copy(ones_v, agg_sh.at[dst_v.at[b]], sem, add=True)
        for b in range(FIRE)
    ]
    for c in copies:
      c.wait()
    return carry

  lax.fori_loop(0, NSUP, body, 0)
  plsc.subcore_barrier()
  pltpu.sync_copy(agg_sh.at[rows], out_hbm.at[cid].at[rows])


@functools.partial(
    pl.kernel,
    out_type=jax.ShapeDtypeStruct((1024, DIM), jnp.float32),
    mesh=_mesh(),
    compiler_params=pltpu.CompilerParams(use_tc_tiling_on_sc=False),
    scratch_types=[
        pltpu.VMEM((32,), jnp.int32),
        pltpu.VMEM((32, DIM), jnp.float32),
        pltpu.SemaphoreType.DMA,
    ],
)
def _sc_gather_targets(table_hbm, idx_hbm, out_hbm, idx_v, rows_v, sem):
  w = _wid()
  base = w * 32
  pltpu.sync_copy(idx_hbm.at[pl.ds(base, 32)], idx_v)
  pltpu.async_copy(table_hbm.at[idx_v], rows_v, sem).wait()
  pltpu.sync_copy(rows_v, out_hbm.at[pl.ds(base, 32)])


def _proj_body(x_ref, w_ref, b_ref, o_ref):
  o_ref[...] = jax.nn.relu(
      jnp.dot(x_ref[...], w_ref[...], preferred_element_type=jnp.float32)
      + b_ref[...])


NB = 2000


def _tc_proj(x, w0, b0):
  full = lambda shape: pl.BlockSpec(shape, lambda i: (0,) * len(shape))
  return pl.pallas_call(
      _proj_body,
      grid=(N // NB,),
      in_specs=[pl.BlockSpec((NB, F_IN), lambda i: (i, 0)),
                full((F_IN, DIM)), full((1, DIM))],
      out_specs=pl.BlockSpec((NB, DIM), lambda i: (i, 0)),
      out_shape=jax.ShapeDtypeStruct((N, DIM), jnp.float32),
  )(x, w0, b0)


EB = 2048


def _msg_body(ea_ref, xj_ref, wn1t_ref, bn1_ref, wn2t_ref, bn2_ref, o_ref):
  h1t = jax.nn.relu(
      jnp.dot(wn1t_ref[...], ea_ref[...], preferred_element_type=jnp.float32)
      + bn1_ref[...])
  ewt = jnp.dot(wn2t_ref[...], h1t, preferred_element_type=jnp.float32,
                precision=lax.Precision.DEFAULT)
  ewt = ewt + bn2_ref[...]
  xjt = xj_ref[...].T
  msgt = xjt[0:1, :] * ewt[0:DIM, :]
  for i in range(1, DIM):
    msgt = msgt + xjt[i:i + 1, :] * ewt[DIM * i:DIM * (i + 1), :]
  o_ref[...] = msgt.T


def _tc_msg(ea8t, xj, wn1t, bn1, wn2t, bn2):
  grid = EP // EB
  full = lambda shape: pl.BlockSpec(shape, lambda i: (0,) * len(shape))
  return pl.pallas_call(
      _msg_body,
      grid=(grid,),
      in_specs=[
          pl.BlockSpec((8, EB), lambda i: (0, i)),
          pl.BlockSpec((EB, DIM), lambda i: (i, 0)),
          full((F_IN, 8)),
          full((F_IN, 1)),
          full((DIM * DIM, F_IN)),
          full((DIM * DIM, 1)),
      ],
      out_specs=pl.BlockSpec((EB, DIM), lambda i: (i, 0)),
      out_shape=jax.ShapeDtypeStruct((EP, DIM), jnp.float32),
  )(ea8t, xj, wn1t, bn1, wn2t, bn2)


def _update_body(agg_ref, deg_ref, h_ref, cb_ref,
                 wr_i, wz_i, wn_i, wr_h, wz_h, wn_h,
                 br_i, bz_i, bn_i, br_h, bz_h, bn_h, o_ref):
  agg = agg_ref[0] + agg_ref[1]
  deg = jnp.maximum(deg_ref[0] + deg_ref[1], 1.0)
  m = jax.nn.relu(agg / deg + cb_ref[...])
  h = h_ref[...]
  dot = lambda a, w: jnp.dot(a, w[...], preferred_element_type=jnp.float32)
  r = jax.nn.sigmoid(dot(m, wr_i) + br_i[...] + dot(h, wr_h) + br_h[...])
  z = jax.nn.sigmoid(dot(m, wz_i) + bz_i[...] + dot(h, wz_h) + bz_h[...])
  n = jnp.tanh(dot(m, wn_i) + bn_i[...] + r * (dot(h, wn_h) + bn_h[...]))
  o_ref[...] = (1.0 - z) * n + z * h


def _tc_update(agg2, deg2, h, cb, gates):
  full = lambda shape: pl.BlockSpec(shape, lambda i: (0,) * len(shape))
  part = pl.BlockSpec((NC, NB, DIM), lambda i: (0, i, 0))
  w16 = full((DIM, DIM))
  b16 = full((1, DIM))
  return pl.pallas_call(
      _update_body,
      grid=(N // NB,),
      in_specs=[part, part,
                pl.BlockSpec((NB, DIM), lambda i: (i, 0)),
                b16, w16, w16, w16, w16, w16, w16,
                b16, b16, b16, b16, b16, b16],
      out_specs=pl.BlockSpec((NB, DIM), lambda i: (i, 0)),
      out_shape=jax.ShapeDtypeStruct((N, DIM), jnp.float32),
  )(agg2, deg2, h, cb, *gates)


def _head_body(n0_ref, w1_ref, b1_ref, w2_ref, b2_ref, o_ref):
  w1c = w1_ref[:DIM, :] + w1_ref[DIM:, :]
  p = jnp.dot(n0_ref[...], w1c, preferred_element_type=jnp.float32)
  p = p + b1_ref[...]
  o_ref[...] = jnp.dot(p, w2_ref[...],
                       preferred_element_type=jnp.float32) + b2_ref[...]


def _tc_head(n0, w1, b1, w2, b2):
  return pl.pallas_call(
      _head_body,
      out_shape=jax.ShapeDtypeStruct((1024, 1), jnp.float32),
  )(n0, w1, b1, w2, b2)


def kernel(x, edge_index, edge_attr, target_indices, W0, b0, Wn1, bn1, Wn2,
           bn2, conv_b, W_ih, W_hh, b_ih, b_hh, W1, b1, W2, b2):
  src = edge_index[0].astype(jnp.int32)
  dst = edge_index[1].astype(jnp.int32)
  atom0 = target_indices[0].astype(jnp.int32)

  pad = EP - E
  src2d = jnp.pad(src, (0, pad)).reshape(EP // CH, CH)
  dst2d = jnp.pad(dst, (0, pad), constant_values=N).reshape(EP // CH, CH)
  ea8t = jnp.pad(edge_attr, ((0, pad), (0, 4))).T
  wn1t = jnp.pad(Wn1, ((0, 4), (0, 0))).T
  wn2t = Wn2.T

  zeros_sub = jnp.zeros((RPS, DIM), jnp.float32)
  ones_ch = jnp.ones((CH, DIM), jnp.float32)

  row = lambda v: v.reshape(1, -1)
  gates = (W_ih[0 * DIM:1 * DIM].T, W_ih[1 * DIM:2 * DIM].T,
           W_ih[2 * DIM:3 * DIM].T, W_hh[0 * DIM:1 * DIM].T,
           W_hh[1 * DIM:2 * DIM].T, W_hh[2 * DIM:3 * DIM].T,
           row(b_ih[0 * DIM:1 * DIM]), row(b_ih[1 * DIM:2 * DIM]),
           row(b_ih[2 * DIM:3 * DIM]), row(b_hh[0 * DIM:1 * DIM]),
           row(b_hh[1 * DIM:2 * DIM]), row(b_hh[2 * DIM:3 * DIM]))

  h = _tc_proj(x, W0, row(b0))
  deg2 = _sc_degree(dst2d, ones_ch, zeros_sub)
  for _ in range(3):
    xj = _sc_gather_edges(h, src2d)
    msg = _tc_msg(ea8t, xj, wn1t, bn1.reshape(-1, 1), wn2t,
                  bn2.reshape(-1, 1))
    agg2 = _sc_scatter_add(msg, dst2d, zeros_sub)
    h = _tc_update(agg2, deg2, h, row(conv_b), gates)

  n0 = _sc_gather_targets(h, atom0)
  return _tc_head(n0, W1, row(b1), W2, row(b2))


# speedup vs baseline: 4.2301x; 1.1406x over previous
import functools

import jax
import jax.numpy as jnp
from jax import lax
from jax.experimental import pallas as pl
from jax.experimental.pallas import tpu as pltpu
from jax.experimental.pallas import tpu_sc as plsc

N = 10000
E = 160000
F_IN = 128
DIM = 16

NC = 2
NS = 16
NW = NC * NS

CH = 128
FIRE = 8
SUP = CH * FIRE
CPW = 40
NSUP = CPW // FIRE
EP = NW * CPW * CH
NP = N + 16
RPS = NP // NS

_mesh = functools.partial(
    plsc.VectorSubcoreMesh,
    core_axis_name="c", subcore_axis_name="s", num_cores=NC, num_subcores=NS,
)


def _wid():
  return lax.axis_index("s") * NC + lax.axis_index("c")


@functools.partial(
    pl.kernel,
    out_type=jax.ShapeDtypeStruct((EP, DIM), jnp.float32),
    mesh=_mesh(),
    compiler_params=pltpu.CompilerParams(use_tc_tiling_on_sc=False,
                                     needs_layout_passes=False),
    scratch_types=[
        pltpu.VMEM((FIRE, CH), jnp.int32),
        pltpu.VMEM((SUP, DIM), jnp.float32),
        pltpu.SemaphoreType.DMA,
    ],
)
def _sc_gather_edges(table_hbm, idx_hbm, out_hbm, idx_v, rows_v, sem):
  w = _wid()
  out_r = out_hbm

  def body(s, carry):
    chunk0 = w * CPW + s * FIRE
    base = chunk0 * CH
    pltpu.sync_copy(idx_hbm.at[pl.ds(chunk0, FIRE)], idx_v)
    copies = [
        pltpu.async_copy(table_hbm.at[idx_v.at[b]],
                         rows_v.at[pl.ds(b * CH, CH)], sem)
        for b in range(FIRE)
    ]
    for c in copies:
      c.wait()
    pltpu.sync_copy(rows_v, out_r.at[pl.ds(base, SUP)])
    return carry

  lax.fori_loop(0, NSUP, body, 0)


@functools.partial(
    pl.kernel,
    out_type=jax.ShapeDtypeStruct((NC, NP, DIM), jnp.float32),
    mesh=_mesh(),
    compiler_params=pltpu.CompilerParams(use_tc_tiling_on_sc=False,
                                     needs_layout_passes=False),
    scratch_types=[
        pltpu.VMEM((FIRE, CH), jnp.int32),
        pltpu.VMEM((FIRE * DIM, 128), jnp.float32),
        pltpu.VMEM((SUP, DIM), jnp.float32),
        pltpu.VMEM_SHARED((NP, DIM), jnp.float32),
        pltpu.SemaphoreType.DMA,
    ],
)
def _sc_scatter_add(msgq_hbm, dst_hbm, zeros_hbm, out_hbm,
                    dst_v, bufq, msg_v, agg_sh, sem):
  cid = lax.axis_index("c")
  sid = lax.axis_index("s")
  w = _wid()
  rows = pl.ds(sid * RPS, RPS)
  pltpu.sync_copy(zeros_hbm, agg_sh.at[rows])
  plsc.subcore_barrier()
  iota = lax.iota(jnp.int32, DIM)

  def body(s, carry):
    chunk0 = w * CPW + s * FIRE
    pltpu.sync_copy(dst_hbm.at[pl.ds(chunk0, FIRE)], dst_v)
    pltpu.sync_copy(msgq_hbm.at[pl.ds(chunk0 * DIM, FIRE * DIM)], bufq)

    def tloop(l, c2):
      lvec = iota * 0 + l
      for b in range(FIRE):
        v = plsc.load_gather(bufq, [DIM * b + iota, lvec])
        plsc.store_scatter(msg_v, [lvec + CH * b, iota], v)
      return c2

    lax.fori_loop(0, CH, tloop, 0)
    copies = [
        pltpu.async_copy(msg_v.at[pl.ds(b * CH, CH)],
                         agg_sh.at[dst_v.at[b]], sem, add=True)
        for b in range(FIRE)
    ]
    for c in copies:
      c.wait()
    return carry

  lax.fori_loop(0, NSUP, body, 0)
  plsc.subcore_barrier()
  pltpu.sync_copy(agg_sh.at[rows], out_hbm.at[cid].at[rows])


@functools.partial(
    pl.kernel,
    out_type=jax.ShapeDtypeStruct((NC, NP, DIM), jnp.float32),
    mesh=_mesh(),
    compiler_params=pltpu.CompilerParams(use_tc_tiling_on_sc=False,
                                     needs_layout_passes=False),
    scratch_types=[
        pltpu.VMEM((FIRE, CH), jnp.int32),
        pltpu.VMEM((CH, DIM), jnp.float32),
        pltpu.VMEM_SHARED((NP, DIM), jnp.float32),
        pltpu.SemaphoreType.DMA,
    ],
)
def _sc_degree(dst_hbm, ones_hbm, zeros_hbm, out_hbm,
               dst_v, ones_v, agg_sh, sem):
  cid = lax.axis_index("c")
  sid = lax.axis_index("s")
  w = _wid()
  rows = pl.ds(sid * RPS, RPS)
  pltpu.sync_copy(zeros_hbm, agg_sh.at[rows])
  pltpu.sync_copy(ones_hbm, ones_v)
  plsc.subcore_barrier()

  def body(s, carry):
    chunk0 = w * CPW + s * FIRE
    pltpu.sync_copy(dst_hbm.at[pl.ds(chunk0, FIRE)], dst_v)
    copies = [
        pltpu.async_copy(ones_v, agg_sh.at[dst_v.at[b]], sem, add=True)
        for b in range(FIRE)
    ]
    for c in copies:
      c.wait()
    return carry

  lax.fori_loop(0, NSUP, body, 0)
  plsc.subcore_barrier()
  pltpu.sync_copy(agg_sh.at[rows], out_hbm.at[cid].at[rows])


@functools.partial(
    pl.kernel,
    out_type=jax.ShapeDtypeStruct((1024, DIM), jnp.float32),
    mesh=_mesh(),
    compiler_params=pltpu.CompilerParams(use_tc_tiling_on_sc=False,
                                     needs_layout_passes=False),
    scratch_types=[
        pltpu.VMEM((32,), jnp.int32),
        pltpu.VMEM((32, DIM), jnp.float32),
        pltpu.SemaphoreType.DMA,
    ],
)
def _sc_gather_targets(table_hbm, idx_hbm, out_hbm, idx_v, rows_v, sem):
  w = _wid()
  base = w * 32
  pltpu.sync_copy(idx_hbm.at[pl.ds(base, 32)], idx_v)
  pltpu.async_copy(table_hbm.at[idx_v], rows_v, sem).wait()
  pltpu.sync_copy(rows_v, out_hbm.at[pl.ds(base, 32)])


def _proj_body(x_ref, w_ref, b_ref, o_ref):
  o_ref[...] = jax.nn.relu(
      jnp.dot(x_ref[...], w_ref[...], preferred_element_type=jnp.float32)
      + b_ref[...])


NB = 2000


def _tc_proj(x, w0, b0):
  full = lambda shape: pl.BlockSpec(shape, lambda i: (0,) * len(shape))
  return pl.pallas_call(
      _proj_body,
      grid=(N // NB,),
      in_specs=[pl.BlockSpec((NB, F_IN), lambda i: (i, 0)),
                full((F_IN, DIM)), full((1, DIM))],
      out_specs=pl.BlockSpec((NB, DIM), lambda i: (i, 0)),
      out_shape=jax.ShapeDtypeStruct((N, DIM), jnp.float32),
  )(x, w0, b0)


EB = 2048


EBP = EB * DIM // 128


def _msg_body(ea_ref, xj_ref, wn1t_ref, bn1_ref, wn2t_ref, bn2_ref, o_ref):
  h1t = jax.nn.relu(
      jnp.dot(wn1t_ref[...], ea_ref[...], preferred_element_type=jnp.float32)
      + bn1_ref[...])
  ewt = jnp.dot(wn2t_ref[...], h1t, preferred_element_type=jnp.float32,
                precision=lax.Precision.DEFAULT)
  ewt = ewt + bn2_ref[...]
  xjq = xj_ref[...].T
  accs = []
  for j in range(8):
    ewj = ewt[:, EBP * j:EBP * (j + 1)]
    acc = xjq[DIM * j:DIM * j + 1, :] * ewj[0:DIM, :]
    for i in range(1, DIM):
      acc = acc + (xjq[DIM * j + i:DIM * j + i + 1, :]
                   * ewj[DIM * i:DIM * (i + 1), :])
    accs.append(acc)
  msgq = jnp.concatenate(accs, axis=0)
  o_ref[...] = jnp.concatenate([msgq[:, :128], msgq[:, 128:]], axis=0)


def _tc_msg(ea8t, xjp, wn1t, bn1, wn2t, bn2):
  grid = EP // EB
  full = lambda shape: pl.BlockSpec(shape, lambda i: (0,) * len(shape))
  return pl.pallas_call(
      _msg_body,
      grid=(grid,),
      in_specs=[
          pl.BlockSpec((8, EB), lambda i: (0, i)),
          pl.BlockSpec((EBP, 128), lambda i: (i, 0)),
          full((F_IN, 8)),
          full((F_IN, 1)),
          full((DIM * DIM, F_IN)),
          full((DIM * DIM, 1)),
      ],
      out_specs=pl.BlockSpec((EBP, 128), lambda i: (i, 0)),
      out_shape=jax.ShapeDtypeStruct((EP * DIM // 128, 128), jnp.float32),
  )(ea8t, xjp, wn1t, bn1, wn2t, bn2)


def _update_body(agg_ref, deg_ref, h_ref, cb_ref,
                 wr_i, wz_i, wn_i, wr_h, wz_h, wn_h,
                 br_i, bz_i, bn_i, br_h, bz_h, bn_h, o_ref):
  agg = agg_ref[0] + agg_ref[1]
  deg = jnp.maximum(deg_ref[0] + deg_ref[1], 1.0)
  m = jax.nn.relu(agg / deg + cb_ref[...])
  h = h_ref[...]
  dot = lambda a, w: jnp.dot(a, w[...], preferred_element_type=jnp.float32)
  r = jax.nn.sigmoid(dot(m, wr_i) + br_i[...] + dot(h, wr_h) + br_h[...])
  z = jax.nn.sigmoid(dot(m, wz_i) + bz_i[...] + dot(h, wz_h) + bz_h[...])
  n = jnp.tanh(dot(m, wn_i) + bn_i[...] + r * (dot(h, wn_h) + bn_h[...]))
  o_ref[...] = (1.0 - z) * n + z * h


def _tc_update(agg2, deg2, h, cb, gates):
  full = lambda shape: pl.BlockSpec(shape, lambda i: (0,) * len(shape))
  part = pl.BlockSpec((NC, NB, DIM), lambda i: (0, i, 0))
  w16 = full((DIM, DIM))
  b16 = full((1, DIM))
  return pl.pallas_call(
      _update_body,
      grid=(N // NB,),
      in_specs=[part, part,
                pl.BlockSpec((NB, DIM), lambda i: (i, 0)),
                b16, w16, w16, w16, w16, w16, w16,
                b16, b16, b16, b16, b16, b16],
      out_specs=pl.BlockSpec((NB, DIM), lambda i: (i, 0)),
      out_shape=jax.ShapeDtypeStruct((N, DIM), jnp.float32),
  )(agg2, deg2, h, cb, *gates)


def _head_body(n0_ref, w1_ref, b1_ref, w2_ref, b2_ref, o_ref):
  w1c = w1_ref[:DIM, :] + w1_ref[DIM:, :]
  p = jnp.dot(n0_ref[...], w1c, preferred_element_type=jnp.float32)
  p = p + b1_ref[...]
  o_ref[...] = jnp.dot(p, w2_ref[...],
                       preferred_element_type=jnp.float32) + b2_ref[...]


def _tc_head(n0, w1, b1, w2, b2):
  return pl.pallas_call(
      _head_body,
      out_shape=jax.ShapeDtypeStruct((1024, 1), jnp.float32),
  )(n0, w1, b1, w2, b2)


def kernel(x, edge_index, edge_attr, target_indices, W0, b0, Wn1, bn1, Wn2,
           bn2, conv_b, W_ih, W_hh, b_ih, b_hh, W1, b1, W2, b2):
  src = edge_index[0].astype(jnp.int32)
  dst = edge_index[1].astype(jnp.int32)
  atom0 = target_indices[0].astype(jnp.int32)

  pad = EP - E
  src2d = jnp.pad(src, (0, pad)).reshape(EP // CH, CH)
  dst2d = (jnp.pad(dst, (0, pad), constant_values=N)
           .reshape(EP // EB, 2, CH, 8).transpose(0, 1, 3, 2)
           .reshape(EP // CH, CH))
  ea8p = (jnp.pad(edge_attr, ((0, pad), (0, 4)))
          .reshape(EP // EB, EBP, 8, 8).transpose(0, 2, 1, 3)
          .reshape(EP, 8).T)
  wn1t = jnp.pad(Wn1, ((0, 4), (0, 0))).T
  wn2t = Wn2.T

  zeros_sub = jnp.zeros((RPS, DIM), jnp.float32)
  ones_ch = jnp.ones((CH, DIM), jnp.float32)

  row = lambda v: v.reshape(1, -1)
  gates = (W_ih[0 * DIM:1 * DIM].T, W_ih[1 * DIM:2 * DIM].T,
           W_ih[2 * DIM:3 * DIM].T, W_hh[0 * DIM:1 * DIM].T,
           W_hh[1 * DIM:2 * DIM].T, W_hh[2 * DIM:3 * DIM].T,
           row(b_ih[0 * DIM:1 * DIM]), row(b_ih[1 * DIM:2 * DIM]),
           row(b_ih[2 * DIM:3 * DIM]), row(b_hh[0 * DIM:1 * DIM]),
           row(b_hh[1 * DIM:2 * DIM]), row(b_hh[2 * DIM:3 * DIM]))

  h = _tc_proj(x, W0, row(b0))
  deg2 = _sc_degree(dst2d, ones_ch, zeros_sub)
  for _ in range(3):
    xj = _sc_gather_edges(h, src2d)
    msgq = _tc_msg(ea8p, xj.reshape(EP * DIM // 128, 128), wn1t,
                   bn1.reshape(-1, 1), wn2t, bn2.reshape(-1, 1))
    agg2 = _sc_scatter_add(msgq, dst2d, zeros_sub)
    h = _tc_update(agg2, deg2, h, row(conv_b), gates)

  n0 = _sc_gather_targets(h, atom0)
  return _tc_head(n0, W1, row(b1), W2, row(b2))


# speedup vs baseline: 5.4347x; 1.2848x over previous
import functools

import jax
import jax.numpy as jnp
from jax import lax
from jax.experimental import pallas as pl
from jax.experimental.pallas import tpu as pltpu
from jax.experimental.pallas import tpu_sc as plsc

N = 10000
E = 160000
F_IN = 128
DIM = 16

NC = 2
NS = 16
NW = NC * NS

CH = 128
FIRE = 8
SUP = CH * FIRE
CPW = 40
NSUP = CPW // FIRE
EP = NW * CPW * CH
NP = N + 16
RPS = NP // NS

_mesh = functools.partial(
    plsc.VectorSubcoreMesh,
    core_axis_name="c", subcore_axis_name="s", num_cores=NC, num_subcores=NS,
)


def _wid():
  return lax.axis_index("s") * NC + lax.axis_index("c")


@functools.partial(
    pl.kernel,
    out_type=jax.ShapeDtypeStruct((EP, DIM), jnp.float32),
    mesh=_mesh(),
    compiler_params=pltpu.CompilerParams(use_tc_tiling_on_sc=False,
                                     needs_layout_passes=False),
    scratch_types=[
        pltpu.VMEM((FIRE, CH), jnp.int32),
        pltpu.VMEM((SUP, DIM), jnp.float32),
        pltpu.SemaphoreType.DMA,
    ],
)
def _sc_gather_edges(table_hbm, idx_hbm, out_hbm, idx_v, rows_v, sem):
  w = _wid()
  out_r = out_hbm

  def body(s, carry):
    chunk0 = w * CPW + s * FIRE
    base = chunk0 * CH
    pltpu.sync_copy(idx_hbm.at[pl.ds(chunk0, FIRE)], idx_v)
    copies = [
        pltpu.async_copy(table_hbm.at[idx_v.at[b]],
                         rows_v.at[pl.ds(b * CH, CH)], sem)
        for b in range(FIRE)
    ]
    for c in copies:
      c.wait()
    pltpu.sync_copy(rows_v, out_r.at[pl.ds(base, SUP)])
    return carry

  lax.fori_loop(0, NSUP, body, 0)


@functools.partial(
    pl.kernel,
    out_type=jax.ShapeDtypeStruct((NC, NP, DIM), jnp.float32),
    mesh=_mesh(),
    compiler_params=pltpu.CompilerParams(use_tc_tiling_on_sc=False,
                                     needs_layout_passes=False),
    scratch_types=[
        pltpu.VMEM((FIRE, CH), jnp.int32),
        pltpu.VMEM((FIRE * DIM, 128), jnp.float32),
        pltpu.VMEM((SUP, DIM), jnp.float32),
        pltpu.VMEM_SHARED((NP, DIM), jnp.float32),
        pltpu.SemaphoreType.DMA,
    ],
)
def _sc_scatter_add(msgq_hbm, dst_hbm, zeros_hbm, out_hbm,
                    dst_v, bufq, msg_v, agg_sh, sem):
  cid = lax.axis_index("c")
  sid = lax.axis_index("s")
  w = _wid()
  rows = pl.ds(sid * RPS, RPS)
  pltpu.sync_copy(zeros_hbm, agg_sh.at[rows])
  plsc.subcore_barrier()
  iota = lax.iota(jnp.int32, DIM)

  def body(s, carry):
    chunk0 = w * CPW + s * FIRE
    pltpu.sync_copy(dst_hbm.at[pl.ds(chunk0, FIRE)], dst_v)
    pltpu.sync_copy(msgq_hbm.at[pl.ds(chunk0 * DIM, FIRE * DIM)], bufq)

    @plsc.parallel_loop(0, CH, unroll=4)
    def tloop(l):
      lvec = iota * 0 + l
      for b in range(FIRE):
        v = plsc.load_gather(bufq, [DIM * b + iota, lvec])
        plsc.store_scatter(msg_v, [lvec + CH * b, iota], v)
    copies = [
        pltpu.async_copy(msg_v.at[pl.ds(b * CH, CH)],
                         agg_sh.at[dst_v.at[b]], sem, add=True)
        for b in range(FIRE)
    ]
    for c in copies:
      c.wait()
    return carry

  lax.fori_loop(0, NSUP, body, 0)
  plsc.subcore_barrier()
  pltpu.sync_copy(agg_sh.at[rows], out_hbm.at[cid].at[rows])


@functools.partial(
    pl.kernel,
    out_type=jax.ShapeDtypeStruct((NC, NP, DIM), jnp.float32),
    mesh=_mesh(),
    compiler_params=pltpu.CompilerParams(use_tc_tiling_on_sc=False,
                                     needs_layout_passes=False),
    scratch_types=[
        pltpu.VMEM((FIRE, CH), jnp.int32),
        pltpu.VMEM((CH, DIM), jnp.float32),
        pltpu.VMEM_SHARED((NP, DIM), jnp.float32),
        pltpu.SemaphoreType.DMA,
    ],
)
def _sc_degree(dst_hbm, ones_hbm, zeros_hbm, out_hbm,
               dst_v, ones_v, agg_sh, sem):
  cid = lax.axis_index("c")
  sid = lax.axis_index("s")
  w = _wid()
  rows = pl.ds(sid * RPS, RPS)
  pltpu.sync_copy(zeros_hbm, agg_sh.at[rows])
  pltpu.sync_copy(ones_hbm, ones_v)
  plsc.subcore_barrier()

  def body(s, carry):
    chunk0 = w * CPW + s * FIRE
    pltpu.sync_copy(dst_hbm.at[pl.ds(chunk0, FIRE)], dst_v)
    copies = [
        pltpu.async_copy(ones_v, agg_sh.at[dst_v.at[b]], sem, add=True)
        for b in range(FIRE)
    ]
    for c in copies:
      c.wait()
    return carry

  lax.fori_loop(0, NSUP, body, 0)
  plsc.subcore_barrier()
  pltpu.sync_copy(agg_sh.at[rows], out_hbm.at[cid].at[rows])


@functools.partial(
    pl.kernel,
    out_type=jax.ShapeDtypeStruct((1024, DIM), jnp.float32),
    mesh=_mesh(),
    compiler_params=pltpu.CompilerParams(use_tc_tiling_on_sc=False,
                                     needs_layout_passes=False),
    scratch_types=[
        pltpu.VMEM((32,), jnp.int32),
        pltpu.VMEM((32, DIM), jnp.float32),
        pltpu.SemaphoreType.DMA,
    ],
)
def _sc_gather_targets(table_hbm, idx_hbm, out_hbm, idx_v, rows_v, sem):
  w = _wid()
  base = w * 32
  pltpu.sync_copy(idx_hbm.at[pl.ds(base, 32)], idx_v)
  pltpu.async_copy(table_hbm.at[idx_v], rows_v, sem).wait()
  pltpu.sync_copy(rows_v, out_hbm.at[pl.ds(base, 32)])


def _proj_body(x_ref, w_ref, b_ref, o_ref):
  o_ref[...] = jax.nn.relu(
      jnp.dot(x_ref[...], w_ref[...], preferred_element_type=jnp.float32)
      + b_ref[...])


NB = 2000


def _tc_proj(x, w0, b0):
  full = lambda shape: pl.BlockSpec(shape, lambda i: (0,) * len(shape))
  return pl.pallas_call(
      _proj_body,
      grid=(N // NB,),
      in_specs=[pl.BlockSpec((NB, F_IN), lambda i: (i, 0)),
                full((F_IN, DIM)), full((1, DIM))],
      out_specs=pl.BlockSpec((NB, DIM), lambda i: (i, 0)),
      out_shape=jax.ShapeDtypeStruct((N, DIM), jnp.float32),
  )(x, w0, b0)


EB = 2048


EBP = EB * DIM // 128


def _msg_body(ea_ref, xj_ref, wn1t_ref, bn1_ref, wn2t_ref, bn2_ref, o_ref):
  h1t = jax.nn.relu(
      jnp.dot(wn1t_ref[...], ea_ref[...], preferred_element_type=jnp.float32)
      + bn1_ref[...])
  ewt = jnp.dot(wn2t_ref[...], h1t, preferred_element_type=jnp.float32,
                precision=lax.Precision.DEFAULT)
  ewt = ewt + bn2_ref[...]
  xjq = xj_ref[...].T
  accs = []
  for j in range(8):
    ewj = ewt[:, EBP * j:EBP * (j + 1)]
    acc = xjq[DIM * j:DIM * j + 1, :] * ewj[0:DIM, :]
    for i in range(1, DIM):
      acc = acc + (xjq[DIM * j + i:DIM * j + i + 1, :]
                   * ewj[DIM * i:DIM * (i + 1), :])
    accs.append(acc)
  msgq = jnp.concatenate(accs, axis=0)
  o_ref[...] = jnp.concatenate([msgq[:, :128], msgq[:, 128:]], axis=0)


def _tc_msg(ea8t, xjp, wn1t, bn1, wn2t, bn2):
  grid = EP // EB
  full = lambda shape: pl.BlockSpec(shape, lambda i: (0,) * len(shape))
  return pl.pallas_call(
      _msg_body,
      grid=(grid,),
      in_specs=[
          pl.BlockSpec((8, EB), lambda i: (0, i)),
          pl.BlockSpec((EBP, 128), lambda i: (i, 0)),
          full((F_IN, 8)),
          full((F_IN, 1)),
          full((DIM * DIM, F_IN)),
          full((DIM * DIM, 1)),
      ],
      out_specs=pl.BlockSpec((EBP, 128), lambda i: (i, 0)),
      out_shape=jax.ShapeDtypeStruct((EP * DIM // 128, 128), jnp.float32),
  )(ea8t, xjp, wn1t, bn1, wn2t, bn2)


def _update_body(agg_ref, deg_ref, h_ref, cb_ref,
                 wr_i, wz_i, wn_i, wr_h, wz_h, wn_h,
                 br_i, bz_i, bn_i, br_h, bz_h, bn_h, o_ref):
  agg = agg_ref[0] + agg_ref[1]
  deg = jnp.maximum(deg_ref[0] + deg_ref[1], 1.0)
  m = jax.nn.relu(agg / deg + cb_ref[...])
  h = h_ref[...]
  dot = lambda a, w: jnp.dot(a, w[...], preferred_element_type=jnp.float32)
  r = jax.nn.sigmoid(dot(m, wr_i) + br_i[...] + dot(h, wr_h) + br_h[...])
  z = jax.nn.sigmoid(dot(m, wz_i) + bz_i[...] + dot(h, wz_h) + bz_h[...])
  n = jnp.tanh(dot(m, wn_i) + bn_i[...] + r * (dot(h, wn_h) + bn_h[...]))
  o_ref[...] = (1.0 - z) * n + z * h


def _tc_update(agg2, deg2, h, cb, gates):
  full = lambda shape: pl.BlockSpec(shape, lambda i: (0,) * len(shape))
  part = pl.BlockSpec((NC, NB, DIM), lambda i: (0, i, 0))
  w16 = full((DIM, DIM))
  b16 = full((1, DIM))
  return pl.pallas_call(
      _update_body,
      grid=(N // NB,),
      in_specs=[part, part,
                pl.BlockSpec((NB, DIM), lambda i: (i, 0)),
                b16, w16, w16, w16, w16, w16, w16,
                b16, b16, b16, b16, b16, b16],
      out_specs=pl.BlockSpec((NB, DIM), lambda i: (i, 0)),
      out_shape=jax.ShapeDtypeStruct((N, DIM), jnp.float32),
  )(agg2, deg2, h, cb, *gates)


def _head_body(n0_ref, w1_ref, b1_ref, w2_ref, b2_ref, o_ref):
  w1c = w1_ref[:DIM, :] + w1_ref[DIM:, :]
  p = jnp.dot(n0_ref[...], w1c, preferred_element_type=jnp.float32)
  p = p + b1_ref[...]
  o_ref[...] = jnp.dot(p, w2_ref[...],
                       preferred_element_type=jnp.float32) + b2_ref[...]


def _tc_head(n0, w1, b1, w2, b2):
  return pl.pallas_call(
      _head_body,
      out_shape=jax.ShapeDtypeStruct((1024, 1), jnp.float32),
  )(n0, w1, b1, w2, b2)


def kernel(x, edge_index, edge_attr, target_indices, W0, b0, Wn1, bn1, Wn2,
           bn2, conv_b, W_ih, W_hh, b_ih, b_hh, W1, b1, W2, b2):
  src = edge_index[0].astype(jnp.int32)
  dst = edge_index[1].astype(jnp.int32)
  atom0 = target_indices[0].astype(jnp.int32)

  pad = EP - E
  src2d = (jnp.pad(src, (0, pad))
           .reshape(EP // EB, 8, EBP).transpose(0, 2, 1)
           .reshape(EP // CH, CH))
  dst2d = (jnp.pad(dst, (0, pad), constant_values=N)
           .reshape(EP // EB, 8, 2, CH).transpose(0, 2, 1, 3)
           .reshape(EP // CH, CH))
  ea8p = jnp.pad(edge_attr, ((0, pad), (0, 4))).T
  wn1t = jnp.pad(Wn1, ((0, 4), (0, 0))).T
  wn2t = Wn2.T

  zeros_sub = jnp.zeros((RPS, DIM), jnp.float32)
  ones_ch = jnp.ones((CH, DIM), jnp.float32)

  row = lambda v: v.reshape(1, -1)
  gates = (W_ih[0 * DIM:1 * DIM].T, W_ih[1 * DIM:2 * DIM].T,
           W_ih[2 * DIM:3 * DIM].T, W_hh[0 * DIM:1 * DIM].T,
           W_hh[1 * DIM:2 * DIM].T, W_hh[2 * DIM:3 * DIM].T,
           row(b_ih[0 * DIM:1 * DIM]), row(b_ih[1 * DIM:2 * DIM]),
           row(b_ih[2 * DIM:3 * DIM]), row(b_hh[0 * DIM:1 * DIM]),
           row(b_hh[1 * DIM:2 * DIM]), row(b_hh[2 * DIM:3 * DIM]))

  h = _tc_proj(x, W0, row(b0))
  deg2 = _sc_degree(dst2d, ones_ch, zeros_sub)
  for _ in range(3):
    xj = _sc_gather_edges(h, src2d)
    msgq = _tc_msg(ea8p, xj.reshape(EP * DIM // 128, 128), wn1t,
                   bn1.reshape(-1, 1), wn2t, bn2.reshape(-1, 1))
    agg2 = _sc_scatter_add(msgq, dst2d, zeros_sub)
    h = _tc_update(agg2, deg2, h, row(conv_b), gates)

  n0 = _sc_gather_targets(h, atom0)
  return _tc_head(n0, W1, row(b1), W2, row(b2))


# speedup vs baseline: 6.0795x; 1.1186x over previous
import functools

import jax
import jax.numpy as jnp
from jax import lax
from jax.experimental import pallas as pl
from jax.experimental.pallas import tpu as pltpu
from jax.experimental.pallas import tpu_sc as plsc

N = 10000
E = 160000
F_IN = 128
DIM = 16

NC = 2
NS = 16
NW = NC * NS

CH = 128
FIRE = 8
SUP = CH * FIRE
CPW = 40
NSUP = CPW // FIRE
EP = NW * CPW * CH
NP = N + 16
RPS = NP // NS

_mesh = functools.partial(
    plsc.VectorSubcoreMesh,
    core_axis_name="c", subcore_axis_name="s", num_cores=NC, num_subcores=NS,
)


def _wid():
  return lax.axis_index("s") * NC + lax.axis_index("c")


@functools.partial(
    pl.kernel,
    out_type=jax.ShapeDtypeStruct((EP, DIM), jnp.float32),
    mesh=_mesh(),
    compiler_params=pltpu.CompilerParams(use_tc_tiling_on_sc=False,
                                     needs_layout_passes=False),
    scratch_types=[
        pltpu.VMEM((FIRE, CH), jnp.int32),
        pltpu.VMEM((SUP, DIM), jnp.float32),
        pltpu.SemaphoreType.DMA,
    ],
)
def _sc_gather_edges(table_hbm, idx_hbm, out_hbm, idx_v, rows_v, sem):
  w = _wid()
  out_r = out_hbm

  def body(s, carry):
    chunk0 = w * CPW + s * FIRE
    base = chunk0 * CH
    pltpu.sync_copy(idx_hbm.at[pl.ds(chunk0, FIRE)], idx_v)
    copies = [
        pltpu.async_copy(table_hbm.at[idx_v.at[b]],
                         rows_v.at[pl.ds(b * CH, CH)], sem)
        for b in range(FIRE)
    ]
    for c in copies:
      c.wait()
    pltpu.sync_copy(rows_v, out_r.at[pl.ds(base, SUP)])
    return carry

  lax.fori_loop(0, NSUP, body, 0)


@functools.partial(
    pl.kernel,
    out_type=jax.ShapeDtypeStruct((NC, NP, DIM), jnp.float32),
    mesh=_mesh(),
    compiler_params=pltpu.CompilerParams(use_tc_tiling_on_sc=False,
                                     needs_layout_passes=False),
    scratch_types=[
        pltpu.VMEM((FIRE, CH), jnp.int32),
        pltpu.VMEM((FIRE * DIM, 128), jnp.float32),
        pltpu.VMEM((SUP, DIM), jnp.float32),
        pltpu.VMEM_SHARED((NP, DIM), jnp.float32),
        pltpu.SemaphoreType.DMA,
    ],
)
def _sc_scatter_add(msgq_hbm, dst_hbm, zeros_hbm, out_hbm,
                    dst_v, bufq, msg_v, agg_sh, sem):
  cid = lax.axis_index("c")
  sid = lax.axis_index("s")
  w = _wid()
  rows = pl.ds(sid * RPS, RPS)
  pltpu.sync_copy(zeros_hbm, agg_sh.at[rows])
  plsc.subcore_barrier()
  iota = lax.iota(jnp.int32, DIM)

  def body(s, carry):
    chunk0 = w * CPW + s * FIRE
    pltpu.sync_copy(dst_hbm.at[pl.ds(chunk0, FIRE)], dst_v)
    pltpu.sync_copy(msgq_hbm.at[pl.ds(chunk0 * DIM, FIRE * DIM)], bufq)

    @plsc.parallel_loop(0, CH, unroll=4)
    def tloop(l):
      lvec = iota * 0 + l
      for b in range(FIRE):
        v = plsc.load_gather(bufq, [DIM * b + iota, lvec])
        plsc.store_scatter(msg_v, [lvec + CH * b, iota], v)
    copies = [
        pltpu.async_copy(msg_v.at[pl.ds(b * CH, CH)],
                         agg_sh.at[dst_v.at[b]], sem, add=True)
        for b in range(FIRE)
    ]
    for c in copies:
      c.wait()
    return carry

  lax.fori_loop(0, NSUP, body, 0)
  plsc.subcore_barrier()
  pltpu.sync_copy(agg_sh.at[rows], out_hbm.at[cid].at[rows])


@functools.partial(
    pl.kernel,
    out_type=jax.ShapeDtypeStruct((NC, NP, DIM), jnp.float32),
    mesh=_mesh(),
    compiler_params=pltpu.CompilerParams(use_tc_tiling_on_sc=False,
                                     needs_layout_passes=False),
    scratch_types=[
        pltpu.VMEM((FIRE, CH), jnp.int32),
        pltpu.VMEM((CH, DIM), jnp.float32),
        pltpu.VMEM_SHARED((NP, DIM), jnp.float32),
        pltpu.SemaphoreType.DMA,
    ],
)
def _sc_degree(dst_hbm, ones_hbm, zeros_hbm, out_hbm,
               dst_v, ones_v, agg_sh, sem):
  cid = lax.axis_index("c")
  sid = lax.axis_index("s")
  w = _wid()
  rows = pl.ds(sid * RPS, RPS)
  pltpu.sync_copy(zeros_hbm, agg_sh.at[rows])
  pltpu.sync_copy(ones_hbm, ones_v)
  plsc.subcore_barrier()

  def body(s, carry):
    chunk0 = w * CPW + s * FIRE
    pltpu.sync_copy(dst_hbm.at[pl.ds(chunk0, FIRE)], dst_v)
    copies = [
        pltpu.async_copy(ones_v, agg_sh.at[dst_v.at[b]], sem, add=True)
        for b in range(FIRE)
    ]
    for c in copies:
      c.wait()
    return carry

  lax.fori_loop(0, NSUP, body, 0)
  plsc.subcore_barrier()
  pltpu.sync_copy(agg_sh.at[rows], out_hbm.at[cid].at[rows])


@functools.partial(
    pl.kernel,
    out_type=jax.ShapeDtypeStruct((1024, DIM), jnp.float32),
    mesh=_mesh(),
    compiler_params=pltpu.CompilerParams(use_tc_tiling_on_sc=False,
                                     needs_layout_passes=False),
    scratch_types=[
        pltpu.VMEM((32,), jnp.int32),
        pltpu.VMEM((32, DIM), jnp.float32),
        pltpu.SemaphoreType.DMA,
    ],
)
def _sc_gather_targets(table_hbm, idx_hbm, out_hbm, idx_v, rows_v, sem):
  w = _wid()
  base = w * 32
  pltpu.sync_copy(idx_hbm.at[pl.ds(base, 32)], idx_v)
  pltpu.async_copy(table_hbm.at[idx_v], rows_v, sem).wait()
  pltpu.sync_copy(rows_v, out_hbm.at[pl.ds(base, 32)])


def _proj_body(x_ref, w_ref, b_ref, o_ref):
  o_ref[...] = jax.nn.relu(
      jnp.dot(x_ref[...], w_ref[...], preferred_element_type=jnp.float32)
      + b_ref[...])


NB = 2000


def _tc_proj(x, w0, b0):
  full = lambda shape: pl.BlockSpec(shape, lambda i: (0,) * len(shape))
  return pl.pallas_call(
      _proj_body,
      grid=(N // NB,),
      in_specs=[pl.BlockSpec((NB, F_IN), lambda i: (i, 0)),
                full((F_IN, DIM)), full((1, DIM))],
      out_specs=pl.BlockSpec((NB, DIM), lambda i: (i, 0)),
      out_shape=jax.ShapeDtypeStruct((N, DIM), jnp.float32),
  )(x, w0, b0)


EB = 4096


EBP = EB * DIM // 128


def _msg_body(ea_ref, xj_ref, wn1t_ref, bn1_ref, wn2t_ref, bn2_ref, o_ref):
  h1t = jax.nn.relu(
      jnp.dot(wn1t_ref[...], ea_ref[...], preferred_element_type=jnp.float32)
      + bn1_ref[...])
  ewt = jnp.dot(wn2t_ref[...], h1t, preferred_element_type=jnp.float32,
                precision=lax.Precision.DEFAULT)
  ewt = ewt + bn2_ref[...]
  xjq = xj_ref[...].T
  accs = []
  for j in range(8):
    ewj = ewt[:, EBP * j:EBP * (j + 1)]
    acc = xjq[DIM * j:DIM * j + 1, :] * ewj[0:DIM, :]
    for i in range(1, DIM):
      acc = acc + (xjq[DIM * j + i:DIM * j + i + 1, :]
                   * ewj[DIM * i:DIM * (i + 1), :])
    accs.append(acc)
  msgq = jnp.concatenate(accs, axis=0)
  o_ref[...] = jnp.concatenate(
      [msgq[:, 128 * hh:128 * (hh + 1)] for hh in range(EBP // 128)], axis=0)


def _tc_msg(ea8t, xjp, wn1t, bn1, wn2t, bn2):
  grid = EP // EB
  full = lambda shape: pl.BlockSpec(shape, lambda i: (0,) * len(shape))
  return pl.pallas_call(
      _msg_body,
      grid=(grid,),
      in_specs=[
          pl.BlockSpec((8, EB), lambda i: (0, i)),
          pl.BlockSpec((EBP, 128), lambda i: (i, 0)),
          full((F_IN, 8)),
          full((F_IN, 1)),
          full((DIM * DIM, F_IN)),
          full((DIM * DIM, 1)),
      ],
      out_specs=pl.BlockSpec((EBP, 128), lambda i: (i, 0)),
      out_shape=jax.ShapeDtypeStruct((EP * DIM // 128, 128), jnp.float32),
  )(ea8t, xjp, wn1t, bn1, wn2t, bn2)


def _update_body(agg_ref, deg_ref, h_ref, cb_ref,
                 wr_i, wz_i, wn_i, wr_h, wz_h, wn_h,
                 br_i, bz_i, bn_i, br_h, bz_h, bn_h, o_ref):
  agg = agg_ref[0] + agg_ref[1]
  deg = jnp.maximum(deg_ref[0] + deg_ref[1], 1.0)
  m = jax.nn.relu(agg / deg + cb_ref[...])
  h = h_ref[...]
  dot = lambda a, w: jnp.dot(a, w[...], preferred_element_type=jnp.float32)
  r = jax.nn.sigmoid(dot(m, wr_i) + br_i[...] + dot(h, wr_h) + br_h[...])
  z = jax.nn.sigmoid(dot(m, wz_i) + bz_i[...] + dot(h, wz_h) + bz_h[...])
  n = jnp.tanh(dot(m, wn_i) + bn_i[...] + r * (dot(h, wn_h) + bn_h[...]))
  o_ref[...] = (1.0 - z) * n + z * h


def _tc_update(agg2, deg2, h, cb, gates):
  full = lambda shape: pl.BlockSpec(shape, lambda i: (0,) * len(shape))
  part = pl.BlockSpec((NC, NB, DIM), lambda i: (0, i, 0))
  w16 = full((DIM, DIM))
  b16 = full((1, DIM))
  return pl.pallas_call(
      _update_body,
      grid=(N // NB,),
      in_specs=[part, part,
                pl.BlockSpec((NB, DIM), lambda i: (i, 0)),
                b16, w16, w16, w16, w16, w16, w16,
                b16, b16, b16, b16, b16, b16],
      out_specs=pl.BlockSpec((NB, DIM), lambda i: (i, 0)),
      out_shape=jax.ShapeDtypeStruct((N, DIM), jnp.float32),
  )(agg2, deg2, h, cb, *gates)


def _head_body(n0_ref, w1_ref, b1_ref, w2_ref, b2_ref, o_ref):
  w1c = w1_ref[:DIM, :] + w1_ref[DIM:, :]
  p = jnp.dot(n0_ref[...], w1c, preferred_element_type=jnp.float32)
  p = p + b1_ref[...]
  o_ref[...] = jnp.dot(p, w2_ref[...],
                       preferred_element_type=jnp.float32) + b2_ref[...]


def _tc_head(n0, w1, b1, w2, b2):
  return pl.pallas_call(
      _head_body,
      out_shape=jax.ShapeDtypeStruct((1024, 1), jnp.float32),
  )(n0, w1, b1, w2, b2)


def kernel(x, edge_index, edge_attr, target_indices, W0, b0, Wn1, bn1, Wn2,
           bn2, conv_b, W_ih, W_hh, b_ih, b_hh, W1, b1, W2, b2):
  src = edge_index[0].astype(jnp.int32)
  dst = edge_index[1].astype(jnp.int32)
  atom0 = target_indices[0].astype(jnp.int32)

  pad = EP - E
  src2d = (jnp.pad(src, (0, pad))
           .reshape(EP // EB, 8, EBP).transpose(0, 2, 1)
           .reshape(EP // CH, CH))
  dst2d = (jnp.pad(dst, (0, pad), constant_values=N)
           .reshape(EP // EB, 8, EBP // CH, CH).transpose(0, 2, 1, 3)
           .reshape(EP // CH, CH))
  ea8p = jnp.pad(edge_attr, ((0, pad), (0, 4))).T
  wn1t = jnp.pad(Wn1, ((0, 4), (0, 0))).T
  wn2t = Wn2.T

  zeros_sub = jnp.zeros((RPS, DIM), jnp.float32)
  ones_ch = jnp.ones((CH, DIM), jnp.float32)

  row = lambda v: v.reshape(1, -1)
  gates = (W_ih[0 * DIM:1 * DIM].T, W_ih[1 * DIM:2 * DIM].T,
           W_ih[2 * DIM:3 * DIM].T, W_hh[0 * DIM:1 * DIM].T,
           W_hh[1 * DIM:2 * DIM].T, W_hh[2 * DIM:3 * DIM].T,
           row(b_ih[0 * DIM:1 * DIM]), row(b_ih[1 * DIM:2 * DIM]),
           row(b_ih[2 * DIM:3 * DIM]), row(b_hh[0 * DIM:1 * DIM]),
           row(b_hh[1 * DIM:2 * DIM]), row(b_hh[2 * DIM:3 * DIM]))

  h = _tc_proj(x, W0, row(b0))
  deg2 = _sc_degree(dst2d, ones_ch, zeros_sub)
  for _ in range(3):
    xj = _sc_gather_edges(h, src2d)
    msgq = _tc_msg(ea8p, xj.reshape(EP * DIM // 128, 128), wn1t,
                   bn1.reshape(-1, 1), wn2t, bn2.reshape(-1, 1))
    agg2 = _sc_scatter_add(msgq, dst2d, zeros_sub)
    h = _tc_update(agg2, deg2, h, row(conv_b), gates)

  n0 = _sc_gather_targets(h, atom0)
  return _tc_head(n0, W1, row(b1), W2, row(b2))


# speedup vs baseline: 6.4499x; 1.0609x over previous
import functools

import jax
import jax.numpy as jnp
from jax import lax
from jax.experimental import pallas as pl
from jax.experimental.pallas import tpu as pltpu
from jax.experimental.pallas import tpu_sc as plsc

N = 10000
E = 160000
F_IN = 128
DIM = 16

NC = 2
NS = 16
NW = NC * NS

CH = 128
FIRE = 8
SUP = CH * FIRE
CPW = 40
NSUP = CPW // FIRE
EP = NW * CPW * CH
NP = N + 16
RPS = NP // NS

_mesh = functools.partial(
    plsc.VectorSubcoreMesh,
    core_axis_name="c", subcore_axis_name="s", num_cores=NC, num_subcores=NS,
)


def _wid():
  return lax.axis_index("s") * NC + lax.axis_index("c")


@functools.partial(
    pl.kernel,
    out_type=jax.ShapeDtypeStruct((EP, DIM), jnp.float32),
    mesh=_mesh(),
    compiler_params=pltpu.CompilerParams(use_tc_tiling_on_sc=False,
                                     needs_layout_passes=False),
    scratch_types=[
        pltpu.VMEM((FIRE, CH), jnp.int32),
        pltpu.VMEM((SUP, DIM), jnp.float32),
        pltpu.SemaphoreType.DMA,
    ],
)
def _sc_gather_edges(table_hbm, idx_hbm, out_hbm, idx_v, rows_v, sem):
  w = _wid()
  out_r = out_hbm

  def body(s, carry):
    chunk0 = w * CPW + s * FIRE
    base = chunk0 * CH
    pltpu.sync_copy(idx_hbm.at[pl.ds(chunk0, FIRE)], idx_v)
    copies = [
        pltpu.async_copy(table_hbm.at[idx_v.at[b]],
                         rows_v.at[pl.ds(b * CH, CH)], sem)
        for b in range(FIRE)
    ]
    for c in copies:
      c.wait()
    pltpu.sync_copy(rows_v, out_r.at[pl.ds(base, SUP)])
    return carry

  lax.fori_loop(0, NSUP, body, 0)


@functools.partial(
    pl.kernel,
    out_type=jax.ShapeDtypeStruct((NC, NP, DIM), jnp.float32),
    mesh=_mesh(),
    compiler_params=pltpu.CompilerParams(use_tc_tiling_on_sc=False,
                                     needs_layout_passes=False),
    scratch_types=[
        pltpu.VMEM((FIRE, CH), jnp.int32),
        pltpu.VMEM((FIRE * DIM, 128), jnp.float32),
        pltpu.VMEM((SUP, DIM), jnp.float32),
        pltpu.VMEM_SHARED((NP, DIM), jnp.float32),
        pltpu.SemaphoreType.DMA,
    ],
)
def _sc_scatter_add(msgq_hbm, dst_hbm, zeros_hbm, out_hbm,
                    dst_v, bufq, msg_v, agg_sh, sem):
  cid = lax.axis_index("c")
  sid = lax.axis_index("s")
  w = _wid()
  rows = pl.ds(sid * RPS, RPS)
  pltpu.sync_copy(zeros_hbm, agg_sh.at[rows])
  plsc.subcore_barrier()
  iota = lax.iota(jnp.int32, DIM)

  def body(s, carry):
    chunk0 = w * CPW + s * FIRE
    pltpu.sync_copy(dst_hbm.at[pl.ds(chunk0, FIRE)], dst_v)
    pltpu.sync_copy(msgq_hbm.at[pl.ds(chunk0 * DIM, FIRE * DIM)], bufq)

    @plsc.parallel_loop(0, CH, unroll=4)
    def tloop(l):
      lvec = iota * 0 + l
      for b in range(FIRE):
        v = plsc.load_gather(bufq, [DIM * b + iota, lvec])
        plsc.store_scatter(msg_v, [lvec + CH * b, iota], v)
    copies = [
        pltpu.async_copy(msg_v.at[pl.ds(b * CH, CH)],
                         agg_sh.at[dst_v.at[b]], sem, add=True)
        for b in range(FIRE)
    ]
    for c in copies:
      c.wait()
    return carry

  lax.fori_loop(0, NSUP, body, 0)
  plsc.subcore_barrier()
  pltpu.sync_copy(agg_sh.at[rows], out_hbm.at[cid].at[rows])


@functools.partial(
    pl.kernel,
    out_type=jax.ShapeDtypeStruct((NC, NP, DIM), jnp.float32),
    mesh=_mesh(),
    compiler_params=pltpu.CompilerParams(use_tc_tiling_on_sc=False,
                                     needs_layout_passes=False),
    scratch_types=[
        pltpu.VMEM((FIRE, CH), jnp.int32),
        pltpu.VMEM((CH, DIM), jnp.float32),
        pltpu.VMEM_SHARED((NP, DIM), jnp.float32),
        pltpu.SemaphoreType.DMA,
    ],
)
def _sc_degree(dst_hbm, ones_hbm, zeros_hbm, out_hbm,
               dst_v, ones_v, agg_sh, sem):
  cid = lax.axis_index("c")
  sid = lax.axis_index("s")
  w = _wid()
  rows = pl.ds(sid * RPS, RPS)
  pltpu.sync_copy(zeros_hbm, agg_sh.at[rows])
  pltpu.sync_copy(ones_hbm, ones_v)
  plsc.subcore_barrier()

  def body(s, carry):
    chunk0 = w * CPW + s * FIRE
    pltpu.sync_copy(dst_hbm.at[pl.ds(chunk0, FIRE)], dst_v)
    copies = [
        pltpu.async_copy(ones_v, agg_sh.at[dst_v.at[b]], sem, add=True)
        for b in range(FIRE)
    ]
    for c in copies:
      c.wait()
    return carry

  lax.fori_loop(0, NSUP, body, 0)
  plsc.subcore_barrier()
  pltpu.sync_copy(agg_sh.at[rows], out_hbm.at[cid].at[rows])


@functools.partial(
    pl.kernel,
    out_type=jax.ShapeDtypeStruct((1024, DIM), jnp.float32),
    mesh=_mesh(),
    compiler_params=pltpu.CompilerParams(use_tc_tiling_on_sc=False,
                                     needs_layout_passes=False),
    scratch_types=[
        pltpu.VMEM((32,), jnp.int32),
        pltpu.VMEM((32, DIM), jnp.float32),
        pltpu.SemaphoreType.DMA,
    ],
)
def _sc_gather_targets(table_hbm, idx_hbm, out_hbm, idx_v, rows_v, sem):
  w = _wid()
  base = w * 32
  pltpu.sync_copy(idx_hbm.at[pl.ds(base, 32)], idx_v)
  pltpu.async_copy(table_hbm.at[idx_v], rows_v, sem).wait()
  pltpu.sync_copy(rows_v, out_hbm.at[pl.ds(base, 32)])


def _proj_body(x_ref, w_ref, b_ref, o_ref):
  o_ref[...] = jax.nn.relu(
      jnp.dot(x_ref[...], w_ref[...], preferred_element_type=jnp.float32)
      + b_ref[...])


NB = 10000


def _tc_proj(x, w0, b0):
  full = lambda shape: pl.BlockSpec(shape, lambda i: (0,) * len(shape))
  return pl.pallas_call(
      _proj_body,
      grid=(N // NB,),
      in_specs=[pl.BlockSpec((NB, F_IN), lambda i: (i, 0)),
                full((F_IN, DIM)), full((1, DIM))],
      out_specs=pl.BlockSpec((NB, DIM), lambda i: (i, 0)),
      out_shape=jax.ShapeDtypeStruct((N, DIM), jnp.float32),
  )(x, w0, b0)


EB = 8192


EBP = EB * DIM // 128


def _msg_body(ea_ref, xj_ref, wn1t_ref, bn1_ref, wn2t_ref, bn2_ref, o_ref):
  h1t = jax.nn.relu(
      jnp.dot(wn1t_ref[...], ea_ref[...], preferred_element_type=jnp.float32)
      + bn1_ref[...])
  ewt = jnp.dot(wn2t_ref[...], h1t, preferred_element_type=jnp.float32,
                precision=lax.Precision.DEFAULT)
  ewt = ewt + bn2_ref[...]
  xjq = xj_ref[...].T
  accs = []
  for j in range(8):
    ewj = ewt[:, EBP * j:EBP * (j + 1)]
    acc = xjq[DIM * j:DIM * j + 1, :] * ewj[0:DIM, :]
    for i in range(1, DIM):
      acc = acc + (xjq[DIM * j + i:DIM * j + i + 1, :]
                   * ewj[DIM * i:DIM * (i + 1), :])
    accs.append(acc)
  msgq = jnp.concatenate(accs, axis=0)
  o_ref[...] = jnp.concatenate(
      [msgq[:, 128 * hh:128 * (hh + 1)] for hh in range(EBP // 128)], axis=0)


def _tc_msg(ea8t, xjp, wn1t, bn1, wn2t, bn2):
  grid = EP // EB
  full = lambda shape: pl.BlockSpec(shape, lambda i: (0,) * len(shape))
  return pl.pallas_call(
      _msg_body,
      grid=(grid,),
      in_specs=[
          pl.BlockSpec((8, EB), lambda i: (0, i)),
          pl.BlockSpec((EBP, 128), lambda i: (i, 0)),
          full((F_IN, 8)),
          full((F_IN, 1)),
          full((DIM * DIM, F_IN)),
          full((DIM * DIM, 1)),
      ],
      out_specs=pl.BlockSpec((EBP, 128), lambda i: (i, 0)),
      out_shape=jax.ShapeDtypeStruct((EP * DIM // 128, 128), jnp.float32),
  )(ea8t, xjp, wn1t, bn1, wn2t, bn2)


def _update_body(agg_ref, deg_ref, h_ref, cb_ref,
                 wr_i, wz_i, wn_i, wr_h, wz_h, wn_h,
                 br_i, bz_i, bn_i, br_h, bz_h, bn_h, o_ref):
  agg = agg_ref[0] + agg_ref[1]
  deg = jnp.maximum(deg_ref[0] + deg_ref[1], 1.0)
  m = jax.nn.relu(agg / deg + cb_ref[...])
  h = h_ref[...]
  dot = lambda a, w: jnp.dot(a, w[...], preferred_element_type=jnp.float32)
  r = jax.nn.sigmoid(dot(m, wr_i) + br_i[...] + dot(h, wr_h) + br_h[...])
  z = jax.nn.sigmoid(dot(m, wz_i) + bz_i[...] + dot(h, wz_h) + bz_h[...])
  n = jnp.tanh(dot(m, wn_i) + bn_i[...] + r * (dot(h, wn_h) + bn_h[...]))
  o_ref[...] = (1.0 - z) * n + z * h


def _tc_update(agg2, deg2, h, cb, gates):
  full = lambda shape: pl.BlockSpec(shape, lambda i: (0,) * len(shape))
  part = pl.BlockSpec((NC, NB, DIM), lambda i: (0, i, 0))
  w16 = full((DIM, DIM))
  b16 = full((1, DIM))
  return pl.pallas_call(
      _update_body,
      grid=(N // NB,),
      in_specs=[part, part,
                pl.BlockSpec((NB, DIM), lambda i: (i, 0)),
                b16, w16, w16, w16, w16, w16, w16,
                b16, b16, b16, b16, b16, b16],
      out_specs=pl.BlockSpec((NB, DIM), lambda i: (i, 0)),
      out_shape=jax.ShapeDtypeStruct((N, DIM), jnp.float32),
  )(agg2, deg2, h, cb, *gates)


def _head_body(n0_ref, w1_ref, b1_ref, w2_ref, b2_ref, o_ref):
  w1c = w1_ref[:DIM, :] + w1_ref[DIM:, :]
  p = jnp.dot(n0_ref[...], w1c, preferred_element_type=jnp.float32)
  p = p + b1_ref[...]
  o_ref[...] = jnp.dot(p, w2_ref[...],
                       preferred_element_type=jnp.float32) + b2_ref[...]


def _tc_head(n0, w1, b1, w2, b2):
  return pl.pallas_call(
      _head_body,
      out_shape=jax.ShapeDtypeStruct((1024, 1), jnp.float32),
  )(n0, w1, b1, w2, b2)


def kernel(x, edge_index, edge_attr, target_indices, W0, b0, Wn1, bn1, Wn2,
           bn2, conv_b, W_ih, W_hh, b_ih, b_hh, W1, b1, W2, b2):
  src = edge_index[0].astype(jnp.int32)
  dst = edge_index[1].astype(jnp.int32)
  atom0 = target_indices[0].astype(jnp.int32)

  pad = EP - E
  src2d = (jnp.pad(src, (0, pad))
           .reshape(EP // EB, 8, EBP).transpose(0, 2, 1)
           .reshape(EP // CH, CH))
  dst2d = (jnp.pad(dst, (0, pad), constant_values=N)
           .reshape(EP // EB, 8, EBP // CH, CH).transpose(0, 2, 1, 3)
           .reshape(EP // CH, CH))
  ea8p = jnp.pad(edge_attr, ((0, pad), (0, 4))).T
  wn1t = jnp.pad(Wn1, ((0, 4), (0, 0))).T
  wn2t = Wn2.T

  zeros_sub = jnp.zeros((RPS, DIM), jnp.float32)
  ones_ch = jnp.ones((CH, DIM), jnp.float32)

  row = lambda v: v.reshape(1, -1)
  gates = (W_ih[0 * DIM:1 * DIM].T, W_ih[1 * DIM:2 * DIM].T,
           W_ih[2 * DIM:3 * DIM].T, W_hh[0 * DIM:1 * DIM].T,
           W_hh[1 * DIM:2 * DIM].T, W_hh[2 * DIM:3 * DIM].T,
           row(b_ih[0 * DIM:1 * DIM]), row(b_ih[1 * DIM:2 * DIM]),
           row(b_ih[2 * DIM:3 * DIM]), row(b_hh[0 * DIM:1 * DIM]),
           row(b_hh[1 * DIM:2 * DIM]), row(b_hh[2 * DIM:3 * DIM]))

  h = _tc_proj(x, W0, row(b0))
  deg2 = _sc_degree(dst2d, ones_ch, zeros_sub)
  for _ in range(3):
    xj = _sc_gather_edges(h, src2d)
    msgq = _tc_msg(ea8p, xj.reshape(EP * DIM // 128, 128), wn1t,
                   bn1.reshape(-1, 1), wn2t, bn2.reshape(-1, 1))
    agg2 = _sc_scatter_add(msgq, dst2d, zeros_sub)
    h = _tc_update(agg2, deg2, h, row(conv_b), gates)

  n0 = _sc_gather_targets(h, atom0)
  return _tc_head(n0, W1, row(b1), W2, row(b2))
